# baseline (speedup 1.0000x reference)
import functools
import math

import jax
import jax.numpy as jnp
from jax import lax
from jax.experimental import pallas as pl
from jax.experimental.pallas import tpu as pltpu
from jax.experimental.pallas import tpu_sc as plsc

_LOG2_HASHMAP_SIZE = 19
_BASE_RESOLUTION = 16
_N_LEVELS = 16


def _grid_metadata():
    per_level_scale = math.exp(
        math.log(2048 * 1 / _BASE_RESOLUTION) / (_N_LEVELS - 1))
    hashmap_size = round(math.pow(2, _LOG2_HASHMAP_SIZE))
    offsets, scales, resolutions = [], [], []
    n_params = 0
    n_levels_normal = 0
    for i in range(_N_LEVELS):
        scale = math.pow(2.0, i * math.log2(per_level_scale)) * _BASE_RESOLUTION - 1.0
        grid_resolution = int(math.ceil(scale)) + 1
        params_in_level = math.pow(grid_resolution, 3)
        params_in_level = math.ceil(params_in_level / 8) * 8
        params_in_level = min(params_in_level, hashmap_size)
        params_in_level = int(params_in_level)
        if math.pow(grid_resolution, 3) <= params_in_level:
            n_levels_normal += 1
        offsets.append(n_params)
        scales.append(scale)
        resolutions.append(grid_resolution)
        n_params += params_in_level
    return offsets, scales, resolutions, n_params, n_levels_normal, hashmap_size


_OFFSETS, _SCALES, _RESOLUTIONS, _N_PARAMS, _N_NORMAL, _HASHMAP_SIZE = _grid_metadata()
_HASH_MASK = _HASHMAP_SIZE - 1
_P1 = 2654435761
_P2 = 805459861


def _body(fvec_hbm, ivec_hbm, grid_hbm, out_hbm, fvec_v, ivec_v, idx0_v,
          idx1_v, vals0_v, vals1_v, out_v, sidx_v, sem):
    cid = lax.axis_index("c")
    sid = lax.axis_index("s")

    @pl.when((cid == 0) & (sid == 0))
    def _():
        pltpu.sync_copy(fvec_hbm, fvec_v)
        pltpu.sync_copy(ivec_hbm, ivec_v)
        xs = fvec_v[pl.ds(0, 16)]
        ys = fvec_v[pl.ds(16, 16)]
        zs = fvec_v[pl.ds(32, 16)]
        scales_v = fvec_v[pl.ds(48, 16)]
        res_v = ivec_v[pl.ds(0, 16)]
        res2_v = ivec_v[pl.ds(16, 16)]
        offs_v = ivec_v[pl.ds(32, 16)]
        lvl = lax.iota(jnp.int32, 16)
        use_hash = lvl >= _N_NORMAL

        def axis_setup(coord):
            pos = scales_v * coord + jnp.float32(0.5)
            base = pos.astype(jnp.int32)
            frac = pos - base.astype(jnp.float32)
            return base, frac

        bx, fx = axis_setup(xs)
        by, fy = axis_setup(ys)
        bz, fz = axis_setup(zs)

        weights = []
        for c in range(8):
            ix, iy, iz = (c >> 2) & 1, (c >> 1) & 1, c & 1
            cx = bx + ix
            cy = by + iy
            cz = bz + iz
            dense = cx + cy * res_v + cz * res2_v
            h = (cx.astype(jnp.uint32)
                 ^ (cy.astype(jnp.uint32) * jnp.uint32(_P1))
                 ^ (cz.astype(jnp.uint32) * jnp.uint32(_P2)))
            h = (h & jnp.uint32(_HASH_MASK)).astype(jnp.int32)
            g = (jnp.where(use_hash, h, dense) + offs_v) * 2
            idx0_v[pl.ds(c * 16, 16)] = g
            idx1_v[pl.ds(c * 16, 16)] = g + 1
            wx = fx if ix else (jnp.float32(1.0) - fx)
            wy = fy if iy else (jnp.float32(1.0) - fy)
            wz = fz if iz else (jnp.float32(1.0) - fz)
            weights.append(wx * wy * wz)

        cp0 = pltpu.async_copy(grid_hbm.at[idx0_v], vals0_v, sem)
        cp1 = pltpu.async_copy(grid_hbm.at[idx1_v], vals1_v, sem)
        cp0.wait()
        cp1.wait()

        acc0 = jnp.zeros((16,), jnp.float32)
        acc1 = jnp.zeros((16,), jnp.float32)
        for c in range(8):
            acc0 = acc0 + weights[c] * vals0_v[pl.ds(c * 16, 16)]
            acc1 = acc1 + weights[c] * vals1_v[pl.ds(c * 16, 16)]

        iota = lax.iota(jnp.int32, 16)
        out_v[pl.ds(0, 16)] = acc0
        out_v[pl.ds(16, 16)] = acc1
        sidx_v[pl.ds(0, 16)] = iota * 2
        sidx_v[pl.ds(16, 16)] = iota * 2 + 1
        pltpu.async_copy(out_v, out_hbm.at[sidx_v], sem).wait()


_hash_encode = functools.partial(
    pl.kernel,
    mesh=plsc.VectorSubcoreMesh(core_axis_name="c", subcore_axis_name="s"),
    out_type=jax.ShapeDtypeStruct((32,), jnp.float32),
    scratch_types=[
        pltpu.VMEM((64,), jnp.float32),
        pltpu.VMEM((48,), jnp.int32),
        pltpu.VMEM((128,), jnp.int32),
        pltpu.VMEM((128,), jnp.int32),
        pltpu.VMEM((128,), jnp.float32),
        pltpu.VMEM((128,), jnp.float32),
        pltpu.VMEM((32,), jnp.float32),
        pltpu.VMEM((32,), jnp.int32),
        pltpu.SemaphoreType.DMA,
    ],
)(_body)

_IVEC_LIST = _RESOLUTIONS + [r * r for r in _RESOLUTIONS] + _OFFSETS


def kernel(inputs, grid):
    fvec = jnp.concatenate(
        [jnp.repeat(inputs.astype(jnp.float32), 16),
         jnp.asarray(_SCALES, dtype=jnp.float32)])
    ivec = jnp.asarray(_IVEC_LIST, dtype=jnp.int32)
    return _hash_encode(fvec, ivec, grid.reshape(-1))


# speedup vs baseline: 127.0222x; 127.0222x over previous
import functools
import math

import jax
import jax.numpy as jnp
from jax import lax
from jax.experimental import pallas as pl
from jax.experimental.pallas import tpu as pltpu
from jax.experimental.pallas import tpu_sc as plsc

_LOG2_HASHMAP_SIZE = 19
_BASE_RESOLUTION = 16
_N_LEVELS = 16


def _grid_metadata():
    per_level_scale = math.exp(
        math.log(2048 * 1 / _BASE_RESOLUTION) / (_N_LEVELS - 1))
    hashmap_size = round(math.pow(2, _LOG2_HASHMAP_SIZE))
    offsets, scales, resolutions = [], [], []
    n_params = 0
    n_levels_normal = 0
    for i in range(_N_LEVELS):
        scale = math.pow(2.0, i * math.log2(per_level_scale)) * _BASE_RESOLUTION - 1.0
        grid_resolution = int(math.ceil(scale)) + 1
        params_in_level = math.pow(grid_resolution, 3)
        params_in_level = math.ceil(params_in_level / 8) * 8
        params_in_level = min(params_in_level, hashmap_size)
        params_in_level = int(params_in_level)
        if math.pow(grid_resolution, 3) <= params_in_level:
            n_levels_normal += 1
        offsets.append(n_params)
        scales.append(scale)
        resolutions.append(grid_resolution)
        n_params += params_in_level
    return offsets, scales, resolutions, n_params, n_levels_normal, hashmap_size


_OFFSETS, _SCALES, _RESOLUTIONS, _N_PARAMS, _N_NORMAL, _HASHMAP_SIZE = _grid_metadata()
_HASH_MASK = _HASHMAP_SIZE - 1
_P1 = 2654435761
_P2 = 805459861

_NBLK = -(-_N_PARAMS // 128)
_NBLK = -(-_NBLK // 8) * 8
_ROWS = _NBLK * 2
_IVEC_LIST = _RESOLUTIONS + [r * r for r in _RESOLUTIONS] + _OFFSETS


def _body(fvec_hbm, ivec_hbm, tab_hbm, out_hbm, fvec_v, ivec_v, ridx0_v,
          ridx1_v, blk0_v, blk1_v, out_v, sidx_v, sem):
    cid = lax.axis_index("c")
    sid = lax.axis_index("s")

    @pl.when((cid == 0) & (sid == 0))
    def _():
        pltpu.sync_copy(fvec_hbm, fvec_v)
        pltpu.sync_copy(ivec_hbm, ivec_v)
        xs = fvec_v[pl.ds(0, 16)]
        ys = fvec_v[pl.ds(16, 16)]
        zs = fvec_v[pl.ds(32, 16)]
        scales_v = fvec_v[pl.ds(48, 16)]
        res_v = ivec_v[pl.ds(0, 16)]
        res2_v = ivec_v[pl.ds(16, 16)]
        offs_v = ivec_v[pl.ds(32, 16)]
        lvl = lax.iota(jnp.int32, 16)
        use_hash = lvl >= _N_NORMAL

        def axis_setup(coord):
            pos = scales_v * coord + jnp.float32(0.5)
            base = pos.astype(jnp.int32)
            frac = pos - base.astype(jnp.float32)
            return base, frac

        bx, fx = axis_setup(xs)
        by, fy = axis_setup(ys)
        bz, fz = axis_setup(zs)

        weights = []
        cols = []
        for c in range(8):
            ix, iy, iz = (c >> 2) & 1, (c >> 1) & 1, c & 1
            cx = bx + ix
            cy = by + iy
            cz = bz + iz
            dense = cx + cy * res_v + cz * res2_v
            h = (cx.astype(jnp.uint32)
                 ^ (cy.astype(jnp.uint32) * jnp.uint32(_P1))
                 ^ (cz.astype(jnp.uint32) * jnp.uint32(_P2)))
            h = (h & jnp.uint32(_HASH_MASK)).astype(jnp.int32)
            idx = jnp.where(use_hash, h, dense) + offs_v
            row0 = (idx >> 7) * 2
            ridx0_v[pl.ds(c * 16, 16)] = row0
            ridx1_v[pl.ds(c * 16, 16)] = row0 + 1
            cols.append(idx & 127)
            wx = fx if ix else (jnp.float32(1.0) - fx)
            wy = fy if iy else (jnp.float32(1.0) - fy)
            wz = fz if iz else (jnp.float32(1.0) - fz)
            weights.append(wx * wy * wz)

        cp0 = pltpu.async_copy(tab_hbm.at[ridx0_v], blk0_v, sem)
        cp1 = pltpu.async_copy(tab_hbm.at[ridx1_v], blk1_v, sem)
        cp0.wait()
        cp1.wait()

        iota = lax.iota(jnp.int32, 16)
        acc0 = jnp.zeros((16,), jnp.float32)
        acc1 = jnp.zeros((16,), jnp.float32)
        for c in range(8):
            rows = iota + (c * 16)
            v0 = plsc.load_gather(blk0_v, [rows, cols[c]])
            v1 = plsc.load_gather(blk1_v, [rows, cols[c]])
            acc0 = acc0 + weights[c] * v0
            acc1 = acc1 + weights[c] * v1

        out_v[pl.ds(0, 16)] = acc0
        out_v[pl.ds(16, 16)] = acc1
        sidx_v[pl.ds(0, 16)] = iota * 2
        sidx_v[pl.ds(16, 16)] = iota * 2 + 1
        pltpu.async_copy(out_v, out_hbm.at[sidx_v], sem).wait()


_hash_encode = functools.partial(
    pl.kernel,
    mesh=plsc.VectorSubcoreMesh(core_axis_name="c", subcore_axis_name="s"),
    out_type=jax.ShapeDtypeStruct((32,), jnp.float32),
    compiler_params=pltpu.CompilerParams(needs_layout_passes=False),
    scratch_types=[
        pltpu.VMEM((64,), jnp.float32),
        pltpu.VMEM((48,), jnp.int32),
        pltpu.VMEM((128,), jnp.int32),
        pltpu.VMEM((128,), jnp.int32),
        pltpu.VMEM((128, 128), jnp.float32),
        pltpu.VMEM((128, 128), jnp.float32),
        pltpu.VMEM((32,), jnp.float32),
        pltpu.VMEM((32,), jnp.int32),
        pltpu.SemaphoreType.DMA,
    ],
)(_body)


def kernel(inputs, grid):
    fvec = jnp.concatenate(
        [jnp.repeat(inputs.astype(jnp.float32), 16),
         jnp.asarray(_SCALES, dtype=jnp.float32)])
    ivec = jnp.asarray(_IVEC_LIST, dtype=jnp.int32)
    gp = jnp.pad(grid, ((0, _NBLK * 128 - _N_PARAMS), (0, 0)))
    tab = jnp.transpose(gp.reshape(_NBLK, 128, 2), (0, 2, 1)).reshape(_ROWS, 128)
    return _hash_encode(fvec, ivec, tab)


# speedup vs baseline: 133.0090x; 1.0471x over previous
import functools
import math

import jax
import jax.numpy as jnp
from jax import lax
from jax.experimental import pallas as pl
from jax.experimental.pallas import tpu as pltpu
from jax.experimental.pallas import tpu_sc as plsc

_LOG2_HASHMAP_SIZE = 19
_BASE_RESOLUTION = 16
_N_LEVELS = 16


def _grid_metadata():
    per_level_scale = math.exp(
        math.log(2048 * 1 / _BASE_RESOLUTION) / (_N_LEVELS - 1))
    hashmap_size = round(math.pow(2, _LOG2_HASHMAP_SIZE))
    offsets, scales, resolutions = [], [], []
    n_params = 0
    n_levels_normal = 0
    for i in range(_N_LEVELS):
        scale = math.pow(2.0, i * math.log2(per_level_scale)) * _BASE_RESOLUTION - 1.0
        grid_resolution = int(math.ceil(scale)) + 1
        params_in_level = math.pow(grid_resolution, 3)
        params_in_level = math.ceil(params_in_level / 8) * 8
        params_in_level = min(params_in_level, hashmap_size)
        params_in_level = int(params_in_level)
        if math.pow(grid_resolution, 3) <= params_in_level:
            n_levels_normal += 1
        offsets.append(n_params)
        scales.append(scale)
        resolutions.append(grid_resolution)
        n_params += params_in_level
    return offsets, scales, resolutions, n_params, n_levels_normal, hashmap_size


_OFFSETS, _SCALES, _RESOLUTIONS, _N_PARAMS, _N_NORMAL, _HASHMAP_SIZE = _grid_metadata()
_HASH_MASK = _HASHMAP_SIZE - 1
_P1 = 2654435761
_P2 = 805459861

_NBLK = -(-(-(-_N_PARAMS // 128)) // 8) * 8
_IVEC_LIST = _RESOLUTIONS + [r * r for r in _RESOLUTIONS] + _OFFSETS


def _body(pvec_hbm, tab_hbm, out_hbm, pvec_v, widx_v, vals_v, out_v,
          sidx_v, sem):
    cid = lax.axis_index("c")
    sid = lax.axis_index("s")

    @pl.when(sid == 0)
    def _():
        pltpu.sync_copy(pvec_hbm, pvec_v)
        res_v = pvec_v[pl.ds(0, 16)]
        res2_v = pvec_v[pl.ds(16, 16)]
        offs_v = pvec_v[pl.ds(32, 16)]
        xs = plsc.bitcast(pvec_v[pl.ds(48, 16)], jnp.float32)
        ys = plsc.bitcast(pvec_v[pl.ds(64, 16)], jnp.float32)
        zs = plsc.bitcast(pvec_v[pl.ds(80, 16)], jnp.float32)
        scales_v = plsc.bitcast(pvec_v[pl.ds(96, 16)], jnp.float32)
        lvl = lax.iota(jnp.int32, 16)
        use_hash = lvl >= _N_NORMAL

        def axis_setup(coord):
            pos = scales_v * coord + jnp.float32(0.5)
            base = pos.astype(jnp.int32)
            frac = pos - base.astype(jnp.float32)
            return base, frac

        bx, fx = axis_setup(xs)
        by, fy = axis_setup(ys)
        bz, fz = axis_setup(zs)

        weights = []
        copies = []
        for c in range(8):
            ix, iy, iz = (c >> 2) & 1, (c >> 1) & 1, c & 1
            cx = bx + ix
            cy = by + iy
            cz = bz + iz
            dense = cx + cy * res_v + cz * res2_v
            h = (cx.astype(jnp.uint32)
                 ^ (cy.astype(jnp.uint32) * jnp.uint32(_P1))
                 ^ (cz.astype(jnp.uint32) * jnp.uint32(_P2)))
            h = (h & jnp.uint32(_HASH_MASK)).astype(jnp.int32)
            idx = jnp.where(use_hash, h, dense) + offs_v
            widx_v[pl.ds(c * 16, 16)] = ((idx >> 7) * 256 + (idx & 127)
                                         + cid * 128)
            copies.append(pltpu.async_copy(
                tab_hbm.at[widx_v.at[pl.ds(c * 16, 16)]],
                vals_v.at[pl.ds(c * 16, 16)], sem))
            wx = fx if ix else (jnp.float32(1.0) - fx)
            wy = fy if iy else (jnp.float32(1.0) - fy)
            wz = fz if iz else (jnp.float32(1.0) - fz)
            weights.append(wx * wy * wz)

        for cp in copies:
            cp.wait()

        iota = lax.iota(jnp.int32, 16)
        acc = jnp.zeros((16,), jnp.float32)
        for c in range(8):
            acc = acc + weights[c] * vals_v[pl.ds(c * 16, 16)]

        out_v[...] = acc
        sidx_v[...] = iota * 2 + cid
        pltpu.async_copy(out_v, out_hbm.at[sidx_v], sem).wait()


_hash_encode = functools.partial(
    pl.kernel,
    mesh=plsc.VectorSubcoreMesh(core_axis_name="c", subcore_axis_name="s"),
    out_type=jax.ShapeDtypeStruct((32,), jnp.float32),
    compiler_params=pltpu.CompilerParams(needs_layout_passes=False),
    scratch_types=[
        pltpu.VMEM((112,), jnp.int32),
        pltpu.VMEM((128,), jnp.int32),
        pltpu.VMEM((128,), jnp.float32),
        pltpu.VMEM((16,), jnp.float32),
        pltpu.VMEM((16,), jnp.int32),
        pltpu.SemaphoreType.DMA,
    ],
)(_body)


def kernel(inputs, grid):
    pvec = jnp.concatenate(
        [jnp.asarray(_IVEC_LIST, dtype=jnp.int32),
         jax.lax.bitcast_convert_type(
             jnp.repeat(inputs.astype(jnp.float32), 16), jnp.int32),
         jax.lax.bitcast_convert_type(
             jnp.asarray(_SCALES, dtype=jnp.float32), jnp.int32)])
    gp = jnp.pad(grid, ((0, _NBLK * 128 - _N_PARAMS), (0, 0)))
    tab = jnp.transpose(gp.reshape(_NBLK, 128, 2), (0, 2, 1)).reshape(_NBLK * 256)
    return _hash_encode(pvec, tab)
